# baseline (speedup 1.0000x reference)
import functools

import jax
import jax.numpy as jnp
from jax import lax
from jax.experimental import pallas as pl
from jax.experimental.pallas import tpu as pltpu
from jax.experimental.pallas import tpu_sc as plsc

B = 256
F = 16384
L = 16
NC = 2
NS = 16
NW = NC * NS
B_PER_W = B // NW
N_PAIRS = B_PER_W // 2
GROUPS = ((0, 1), (2, 3))
C = 4096
UNROLL = 8
HI_MASK = -65536


def _body(n_chunks, pinp_hbm, packed_hbm, vals_hbm, bias_idx_hbm,
          bias_val_hbm, out_hbm, pinp0_v, pinp1_v,
          acc0_v, acc1_v, acc2_v, acc3_v, pk0_v, vl0_v, pk1_v, vl1_v,
          bi_v, bv_v, sem0, sem1):
  pinp_refs = (pinp0_v, pinp1_v)
  acc_refs = (acc0_v, acc1_v, acc2_v, acc3_v)
  pk_refs = (pk0_v, pk1_v)
  vl_refs = (vl0_v, vl1_v)
  sems = (sem0, sem1)
  cid = lax.axis_index("c")
  sid = lax.axis_index("s")
  wid = sid * NC + cid
  b_base = wid * B_PER_W
  pair_base = wid * N_PAIRS

  pltpu.sync_copy(bias_idx_hbm, bi_v)
  pltpu.sync_copy(bias_val_hbm, bv_v)
  n_bias_vec = bi_v.shape[0] // L

  def start_fetch(ch, slot):
    pltpu.async_copy(packed_hbm.at[pl.ds(ch * C, C)], pk_refs[slot],
                     sems[slot])
    pltpu.async_copy(vals_hbm.at[pl.ds(ch * C, C)], vl_refs[slot],
                     sems[slot])

  def wait_fetch(ch, slot):
    pltpu.make_async_copy(packed_hbm.at[pl.ds(ch * C, C)], pk_refs[slot],
                          sems[slot]).wait()
    pltpu.make_async_copy(vals_hbm.at[pl.ds(ch * C, C)], vl_refs[slot],
                          sems[slot]).wait()

  n_dma_pairs = n_chunks // 2
  assert n_dma_pairs * 2 == n_chunks

  for group in GROUPS:
    def process(slot):
      pk_ref, vl_ref = pk_refs[slot], vl_refs[slot]

      @plsc.parallel_loop(0, C, step=L, unroll=UNROLL)
      def _(off):
        pk = pk_ref[pl.ds(off, L)]
        v = vl_ref[pl.ds(off, L)]
        r = pk >> 14
        c = pk & (F - 1)
        for g in range(len(group)):
          w = plsc.load_gather(pinp_refs[g], [c])
          x_hi = plsc.bitcast(w & HI_MASK, jnp.float32)
          x_lo = plsc.bitcast(w << 16, jnp.float32)
          plsc.addupdate_scatter(acc_refs[2 * g], [r], x_hi * v)
          plsc.addupdate_scatter(acc_refs[2 * g + 1], [r], x_lo * v)

    start_fetch(0, 0)

    for g, pair in enumerate(group):
      pltpu.sync_copy(pinp_hbm.at[pair_base + pair], pinp_refs[g])

    for a in range(4):
      @plsc.parallel_loop(0, F, step=L, unroll=UNROLL)
      def _(off, a=a):
        acc_refs[a][pl.ds(off, L)] = jnp.zeros((L,), jnp.float32)

      def bias_body(i, _, a=a):
        idx = bi_v[pl.ds(i * L, L)]
        val = bv_v[pl.ds(i * L, L)]
        plsc.addupdate_scatter(acc_refs[a], [idx], val)
        return 0
      lax.fori_loop(0, n_bias_vec, bias_body, 0)

    def pair_body(p, _, process=process):
      ch0 = 2 * p
      start_fetch(ch0 + 1, 1)
      wait_fetch(ch0, 0)
      process(0)

      @pl.when(p < n_dma_pairs - 1)
      def _():
        start_fetch(ch0 + 2, 0)
      wait_fetch(ch0 + 1, 1)
      process(1)
      return 0
    lax.fori_loop(0, n_dma_pairs, pair_body, 0)

    for g, pair in enumerate(group):
      pltpu.sync_copy(acc_refs[2 * g], out_hbm.at[b_base + 2 * pair])
      pltpu.sync_copy(acc_refs[2 * g + 1], out_hbm.at[b_base + 2 * pair + 1])


def kernel(input, weight_indices, weight_values, bias_indices, bias_values):
  rows = weight_indices[0].astype(jnp.int32)
  cols = weight_indices[1].astype(jnp.int32)
  packed = rows * F + cols
  vals = weight_values.astype(jnp.float32)

  u = lax.bitcast_convert_type(
      input.astype(jnp.bfloat16), jnp.uint16).astype(jnp.uint32)
  pinp = ((u[0::2] << 16) | u[1::2]).astype(jnp.int32)

  nnz = packed.shape[0]
  n_chunks = -(-nnz // C)
  if n_chunks % 2:
    n_chunks += 1
  pad = n_chunks * C - nnz
  packed = jnp.concatenate([packed, jnp.zeros((pad,), jnp.int32)])
  vals = jnp.concatenate([vals, jnp.zeros((pad,), jnp.float32)])

  bias_idx = bias_indices.astype(jnp.int32)
  bn = bias_idx.shape[0]
  bias_pad = -(-bn // L) * L - bn
  bias_idx = jnp.concatenate([bias_idx, jnp.zeros((bias_pad,), jnp.int32)])
  bias_val = jnp.concatenate(
      [bias_values.astype(jnp.float32), jnp.zeros((bias_pad,), jnp.float32)])

  mesh = plsc.VectorSubcoreMesh(core_axis_name="c", subcore_axis_name="s")
  run = pl.kernel(
      functools.partial(_body, n_chunks),
      out_type=jax.ShapeDtypeStruct((B, F), jnp.float32),
      mesh=mesh,
      compiler_params=pltpu.CompilerParams(needs_layout_passes=False),
      scratch_types=[
          pltpu.VMEM((F,), jnp.int32),
          pltpu.VMEM((F,), jnp.int32),
          pltpu.VMEM((F,), jnp.float32),
          pltpu.VMEM((F,), jnp.float32),
          pltpu.VMEM((F,), jnp.float32),
          pltpu.VMEM((F,), jnp.float32),
          pltpu.VMEM((C,), jnp.int32),
          pltpu.VMEM((C,), jnp.float32),
          pltpu.VMEM((C,), jnp.int32),
          pltpu.VMEM((C,), jnp.float32),
          pltpu.VMEM((bias_idx.shape[0],), jnp.int32),
          pltpu.VMEM((bias_idx.shape[0],), jnp.float32),
          pltpu.SemaphoreType.DMA,
          pltpu.SemaphoreType.DMA,
      ],
  )
  return run(pinp, packed, vals, bias_idx, bias_val)


# speedup vs baseline: 1.0853x; 1.0853x over previous
import functools

import jax
import jax.numpy as jnp
from jax import lax
from jax.experimental import pallas as pl
from jax.experimental.pallas import tpu as pltpu
from jax.experimental.pallas import tpu_sc as plsc

B = 256
F = 16384
L = 16
NC = 2
NS = 16
NW = NC * NS
B_PER_W = B // NW
BATCHES = ((0, 1, 2), (3, 4, 5), (6, 7))
C = 4096
UNROLL = 4


def _body(n_chunks, inp_hbm, packed_hbm, vals_hbm, bias_idx_hbm,
          bias_val_hbm, out_hbm, inp0_v, inp1_v, inp2_v,
          acc0_v, acc1_v, acc2_v, pk0_v, vl0_v, pk1_v, vl1_v,
          bi_v, bv_v, sem0, sem1):
  inp_refs = (inp0_v, inp1_v, inp2_v)
  acc_refs = (acc0_v, acc1_v, acc2_v)
  pk_refs = (pk0_v, pk1_v)
  vl_refs = (vl0_v, vl1_v)
  sems = (sem0, sem1)
  cid = lax.axis_index("c")
  sid = lax.axis_index("s")
  wid = sid * NC + cid
  b_base = wid * B_PER_W

  pltpu.sync_copy(bias_idx_hbm, bi_v)
  pltpu.sync_copy(bias_val_hbm, bv_v)
  n_bias_vec = bi_v.shape[0] // L

  def start_fetch(ch, slot):
    pltpu.async_copy(packed_hbm.at[pl.ds(ch * C, C)], pk_refs[slot],
                     sems[slot])
    pltpu.async_copy(vals_hbm.at[pl.ds(ch * C, C)], vl_refs[slot],
                     sems[slot])

  def wait_fetch(ch, slot):
    pltpu.make_async_copy(packed_hbm.at[pl.ds(ch * C, C)], pk_refs[slot],
                          sems[slot]).wait()
    pltpu.make_async_copy(vals_hbm.at[pl.ds(ch * C, C)], vl_refs[slot],
                          sems[slot]).wait()

  n_pairs = n_chunks // 2
  assert n_pairs * 2 == n_chunks

  for group in BATCHES:
    nb = len(group)

    def process(slot, nb=nb):
      pk_ref, vl_ref = pk_refs[slot], vl_refs[slot]

      @plsc.parallel_loop(0, C, step=L, unroll=UNROLL)
      def _(off):
        pk = pk_ref[pl.ds(off, L)]
        v = vl_ref[pl.ds(off, L)]
        r = pk >> 14
        c = pk & (F - 1)
        for j in range(nb):
          x = plsc.load_gather(inp_refs[j], [c])
          plsc.addupdate_scatter(acc_refs[j], [r], x * v)

    start_fetch(0, 0)

    for j, db in enumerate(group):
      pltpu.sync_copy(inp_hbm.at[b_base + db], inp_refs[j])

      @plsc.parallel_loop(0, F, step=L, unroll=UNROLL)
      def _(off, j=j):
        acc_refs[j][pl.ds(off, L)] = jnp.zeros((L,), jnp.float32)

      def bias_body(i, _, j=j):
        idx = bi_v[pl.ds(i * L, L)]
        val = bv_v[pl.ds(i * L, L)]
        plsc.addupdate_scatter(acc_refs[j], [idx], val)
        return 0
      lax.fori_loop(0, n_bias_vec, bias_body, 0)

    def pair_body(p, _, process=process):
      ch0 = 2 * p
      start_fetch(ch0 + 1, 1)
      wait_fetch(ch0, 0)
      process(0)

      @pl.when(p < n_pairs - 1)
      def _():
        start_fetch(ch0 + 2, 0)
      wait_fetch(ch0 + 1, 1)
      process(1)
      return 0
    lax.fori_loop(0, n_pairs, pair_body, 0)

    for j, db in enumerate(group):
      pltpu.sync_copy(acc_refs[j], out_hbm.at[b_base + db])


def kernel(input, weight_indices, weight_values, bias_indices, bias_values):
  rows = weight_indices[0].astype(jnp.int32)
  cols = weight_indices[1].astype(jnp.int32)
  packed = rows * F + cols
  vals = weight_values.astype(jnp.float32)

  nnz = packed.shape[0]
  n_chunks = -(-nnz // C)
  if n_chunks % 2:
    n_chunks += 1
  pad = n_chunks * C - nnz
  packed = jnp.concatenate([packed, jnp.zeros((pad,), jnp.int32)])
  vals = jnp.concatenate([vals, jnp.zeros((pad,), jnp.float32)])

  bias_idx = bias_indices.astype(jnp.int32)
  bn = bias_idx.shape[0]
  bias_pad = -(-bn // L) * L - bn
  bias_idx = jnp.concatenate([bias_idx, jnp.zeros((bias_pad,), jnp.int32)])
  bias_val = jnp.concatenate(
      [bias_values.astype(jnp.float32), jnp.zeros((bias_pad,), jnp.float32)])

  mesh = plsc.VectorSubcoreMesh(core_axis_name="c", subcore_axis_name="s")
  run = pl.kernel(
      functools.partial(_body, n_chunks),
      out_type=jax.ShapeDtypeStruct((B, F), jnp.float32),
      mesh=mesh,
      compiler_params=pltpu.CompilerParams(needs_layout_passes=False),
      scratch_types=[
          pltpu.VMEM((F,), jnp.float32),
          pltpu.VMEM((F,), jnp.float32),
          pltpu.VMEM((F,), jnp.float32),
          pltpu.VMEM((F,), jnp.float32),
          pltpu.VMEM((F,), jnp.float32),
          pltpu.VMEM((F,), jnp.float32),
          pltpu.VMEM((C,), jnp.int32),
          pltpu.VMEM((C,), jnp.float32),
          pltpu.VMEM((C,), jnp.int32),
          pltpu.VMEM((C,), jnp.float32),
          pltpu.VMEM((bias_idx.shape[0],), jnp.int32),
          pltpu.VMEM((bias_idx.shape[0],), jnp.float32),
          pltpu.SemaphoreType.DMA,
          pltpu.SemaphoreType.DMA,
      ],
  )
  return run(input, packed, vals, bias_idx, bias_val)
